# Initial kernel scaffold; baseline (speedup 1.0000x reference)
import jax
import jax.numpy as jnp
from jax.experimental import pallas as pl


def kernel(x, edge_index, edge_attr, batch, params):
    raise NotImplementedError("write your pallas kernel here")


# speedup vs baseline: 2.6270x; 2.6270x over previous
import functools

import jax
import jax.numpy as jnp
from jax import lax
from jax.experimental import pallas as pl
from jax.experimental.pallas import tpu as pltpu
from jax.experimental.pallas import tpu_sc as plsc

_N = 10000
_E = 320000
_D = 128
_G = 64
_NC = 2
_NS = 16
_NW = _NC * _NS
_EPW = _E // _NW
_K = 80
_NCHUNK = _EPW // _K
_NP = 10240
_RPS = _NP // _NS


def _bf16_round(v):
    u = lax.bitcast_convert_type(v, jnp.uint32)
    r = (u + jnp.uint32(0x7FFF) + ((u >> jnp.uint32(16)) & jnp.uint32(1)))
    r = r & jnp.uint32(0xFFFF0000)
    return lax.bitcast_convert_type(r, jnp.float32)


def _sc_aggregate(x, src, dst, attr, wb):
    mesh = plsc.VectorSubcoreMesh(core_axis_name="c", subcore_axis_name="s")

    @functools.partial(
        pl.kernel,
        out_type=jax.ShapeDtypeStruct((_NC * _NP, _D), jnp.float32),
        mesh=mesh,
        scratch_types=[
            pltpu.VMEM((_K,), jnp.int32),
            pltpu.VMEM((_K,), jnp.int32),
            pltpu.VMEM((_K, _D), jnp.float32),
            pltpu.VMEM((2 * _K,), jnp.float32),
            pltpu.VMEM((8, _D), jnp.float32),
            pltpu.VMEM((128, _D), jnp.float32),
            pltpu.VMEM_SHARED((_NP, _D), jnp.float32),
        ],
    )
    def k(x_hbm, src_hbm, dst_hbm, attr_hbm, wb_hbm, out_hbm,
          src_v, dst_v, rows_v, attr_v, wb_v, zero_v, acc_sh):
        c = lax.axis_index("c")
        s = lax.axis_index("s")
        wid = c * _NS + s

        pltpu.sync_copy(wb_hbm, wb_v)

        @pl.loop(0, 128)
        def _(r):
            for g in range(8):
                zero_v[r, pl.ds(g * 16, 16)] = jnp.zeros((16,), jnp.float32)

        @pl.loop(0, _RPS // 128)
        def _(t):
            pltpu.sync_copy(zero_v, acc_sh.at[pl.ds(s * _RPS + t * 128, 128)])

        plsc.subcore_barrier()

        w0 = [_bf16_round(wb_v[0, pl.ds(g * 16, 16)]) for g in range(8)]
        w1 = [_bf16_round(wb_v[1, pl.ds(g * 16, 16)]) for g in range(8)]
        bb = [wb_v[2, pl.ds(g * 16, 16)] for g in range(8)]

        base0 = wid * _EPW

        @pl.loop(0, _NCHUNK)
        def _(i):
            base = base0 + i * _K
            pltpu.sync_copy(src_hbm.at[pl.ds(base, _K)], src_v)
            pltpu.sync_copy(dst_hbm.at[pl.ds(base, _K)], dst_v)
            pltpu.sync_copy(attr_hbm.at[pl.ds(2 * base, 2 * _K)], attr_v)
            pltpu.sync_copy(x_hbm.at[src_v], rows_v)

            @pl.loop(0, _K // 8)
            def _(t):
                av = _bf16_round(attr_v[pl.ds(t * 16, 16)])
                for j2 in range(8):
                    a0 = av[2 * j2]
                    a1 = av[2 * j2 + 1]
                    j = t * 8 + j2
                    for g in range(8):
                        sl = pl.ds(g * 16, 16)
                        v = rows_v[j, sl] + (a0 * w0[g] + a1 * w1[g] + bb[g])
                        rows_v[j, sl] = jnp.maximum(v, 0.0)

            pltpu.sync_copy(rows_v, acc_sh.at[dst_v], add=True)

        plsc.subcore_barrier()
        pltpu.sync_copy(acc_sh.at[pl.ds(s * _RPS, _RPS)],
                        out_hbm.at[pl.ds(c * _NP + s * _RPS, _RPS)])

    return k(x, src, dst, attr, wb)


def _mm(a, b):
    return jnp.dot(a, b, preferred_element_type=jnp.float32)


def _tree8(a):
    r = a[0:4] + a[4:8]
    r = r[0:2] + r[2:4]
    return r[0:1] + r[1:2]


def _sum0_like_xla(scr):
    nv = _N // 16

    def chunk(v0):
        def step(i, acc):
            return acc + scr[pl.ds((v0 + i) * 8, 8), :]
        return lax.fori_loop(1, nv, step, scr[pl.ds(v0 * 8, 8), :])

    return _tree8(chunk(0)) + _tree8(chunk(nv))


def _lin_bn_relu(x, w, b, gamma, beta, scr):
    z = _mm(x, w) + b
    scr[...] = z
    m = _sum0_like_xla(scr) / jnp.float32(_N)
    zc = z - m
    scr[...] = zc * zc
    v = _sum0_like_xla(scr) / jnp.float32(_N)
    return jnp.maximum(zc * lax.rsqrt(v + 1e-5) * gamma + beta, 0.0)


def _lin_bn_relu_small(x, w, b, gamma, beta):
    z = _mm(x, w) + b
    m = jnp.mean(z, axis=0, keepdims=True)
    v = jnp.mean((z - m) ** 2, axis=0, keepdims=True)
    return jnp.maximum((z - m) * lax.rsqrt(v + 1e-5) * gamma + beta, 0.0)


def _tc_layer(h_prev, agg, eps, mlp, residual):
    def body(h_ref, a_ref, eps_ref, w1, b1, g1, be1, w2, b2, g2, be2,
             w3, b3, o_ref, scr):
        t = (1.0 + eps_ref[0, 0]) * h_ref[...] + a_ref[0] + a_ref[1]
        t = _lin_bn_relu(t, w1[...], b1[...], g1[...], be1[...], scr)
        t = _lin_bn_relu(t, w2[...], b2[...], g2[...], be2[...], scr)
        t = _mm(t, w3[...]) + b3[...]
        t = jnp.maximum(t, 0.0)
        if residual:
            t = t + h_ref[...]
        o_ref[...] = t

    p1, p2, p3 = mlp
    args = (h_prev, agg, eps.reshape(1, 1),
            p1["W"], p1["b"].reshape(1, _D), p1["gamma"].reshape(1, _D),
            p1["beta"].reshape(1, _D),
            p2["W"], p2["b"].reshape(1, _D), p2["gamma"].reshape(1, _D),
            p2["beta"].reshape(1, _D),
            p3["W"], p3["b"].reshape(1, _D))
    return pl.pallas_call(
        body,
        out_shape=jax.ShapeDtypeStruct((_N, _D), jnp.float32),
        scratch_shapes=[pltpu.VMEM((_N, _D), jnp.float32)],
    )(*args)


def _tc_readout(reps, batch_f, linears):
    def body(*refs):
        r_refs = refs[:4]
        b_ref = refs[4]
        o_ref = refs[-1]
        labels = lax.broadcasted_iota(jnp.int32, (1, _G), 1)
        oh = (b_ref[...] == labels).astype(jnp.float32)
        total = None
        for i in range(4):
            w1, b1, g1, be1, w2, b2 = refs[5 + 6 * i:5 + 6 * (i + 1)]
            pooled = lax.dot_general(
                oh, r_refs[i][...],
                dimension_numbers=(((0,), (0,)), ((), ())),
                preferred_element_type=jnp.float32,
                precision=lax.Precision.HIGHEST)
            z = _lin_bn_relu_small(pooled, w1[...], b1[...], g1[...], be1[...])
            z = _mm(z, w2[...]) + b2[...]
            total = z if total is None else total + z
        o_ref[...] = total

    args = list(reps) + [batch_f]
    for lin in linears:
        p1, p2 = lin
        args += [p1["W"], p1["b"].reshape(1, _D), p1["gamma"].reshape(1, _D),
                 p1["beta"].reshape(1, _D), p2["W"], p2["b"].reshape(1, _D)]
    return pl.pallas_call(
        body,
        out_shape=jax.ShapeDtypeStruct((_G, _D), jnp.float32),
    )(*args)


def kernel(x, edge_index, edge_attr, batch, params):
    src = edge_index[0]
    dst = edge_index[1]
    attr_flat = edge_attr.reshape(2 * _E)
    batch_i = batch.reshape(_N, 1)

    reps = []
    h = x
    reps.append(h)
    for li, cp in enumerate(params["convs"]):
        wb = jnp.concatenate(
            [cp["lin_W"], cp["lin_b"].reshape(1, _D),
             jnp.zeros((5, _D), jnp.float32)], axis=0)
        agg = _sc_aggregate(h, src, dst, attr_flat, wb)
        agg = agg.reshape(2, _NP, _D)[:, :_N, :]
        h = _tc_layer(h, agg, cp["eps"], cp["mlp"], residual=(li > 0))
        reps.append(h)

    return _tc_readout(reps, batch_i, params["linears"])
